# baseline (speedup 1.0000x reference)
import numpy as np

import jax
import jax.numpy as jnp
from jax import lax
from jax.experimental import pallas as pl
from jax.experimental.pallas import tpu as pltpu
from jax.experimental.pallas import tpu_sc as plsc

_NUM_TRAIN_TIMESTEPS = 1000
_BETA_START = 0.0001
_BETA_END = 0.02
_PAD = 128


def _schedule_table():
    betas = np.linspace(_BETA_START, _BETA_END, _NUM_TRAIN_TIMESTEPS,
                        dtype=np.float32)
    alphas_cumprod = np.cumprod(1.0 - betas, axis=0, dtype=np.float32)
    table = np.zeros((_NUM_TRAIN_TIMESTEPS, _PAD), dtype=np.float32)
    table[:, 0] = np.sqrt(alphas_cumprod)
    table[:, 1] = np.sqrt(1.0 - alphas_cumprod)
    return table


_TABLE = _schedule_table()


def _sc_gather_body(table_hbm, idx_hbm, out_hbm, idx_v, rows_v, sem):
    wid0 = (lax.axis_index("c") == 0) & (lax.axis_index("s") == 0)

    @pl.when(wid0)
    def _():
        pltpu.sync_copy(idx_hbm, idx_v)
        pltpu.async_copy(table_hbm.at[idx_v], rows_v, sem).wait()
        pltpu.sync_copy(rows_v, out_hbm)


def _sc_gather(table, idx):
    b = idx.shape[0]
    return pl.kernel(
        _sc_gather_body,
        out_type=jax.ShapeDtypeStruct((b, _PAD), jnp.float32),
        mesh=plsc.VectorSubcoreMesh(core_axis_name="c", subcore_axis_name="s"),
        scratch_types=[
            pltpu.VMEM((b,), jnp.int32),
            pltpu.VMEM((b, _PAD), jnp.float32),
            pltpu.SemaphoreType.DMA,
        ],
    )(table, idx)


def _fma_body(g_ref, x_ref, n_ref, o_ref):
    b = pl.program_id(0)
    a = g_ref[b, 0]
    s = g_ref[b, 1]
    o_ref[...] = x_ref[...] * a + n_ref[...] * s


def kernel(x_0, noise, timesteps):
    B, N, C = x_0.shape
    gathered = _sc_gather(jnp.asarray(_TABLE), timesteps.astype(jnp.int32))
    spec = pl.BlockSpec((1, N, C), lambda i, *_: (i, 0, 0))
    return pl.pallas_call(
        _fma_body,
        grid_spec=pltpu.PrefetchScalarGridSpec(
            num_scalar_prefetch=1,
            grid=(B,),
            in_specs=[spec, spec],
            out_specs=spec,
        ),
        out_shape=jax.ShapeDtypeStruct((B, N, C), x_0.dtype),
        compiler_params=pltpu.CompilerParams(
            dimension_semantics=("parallel",),
        ),
    )(gathered, x_0, noise)


# speedup vs baseline: 1.0581x; 1.0581x over previous
import numpy as np

import jax
import jax.numpy as jnp
from jax.experimental import pallas as pl
from jax.experimental.pallas import tpu as pltpu

_NUM_TRAIN_TIMESTEPS = 1000
_BETA_START = 0.0001
_BETA_END = 0.02


def _schedule_tables():
    betas = np.linspace(_BETA_START, _BETA_END, _NUM_TRAIN_TIMESTEPS,
                        dtype=np.float32)
    alphas_cumprod = np.cumprod(1.0 - betas, axis=0, dtype=np.float32)
    sqrt_a = np.sqrt(alphas_cumprod).astype(np.float32)
    sqrt_oma = np.sqrt(1.0 - alphas_cumprod).astype(np.float32)
    return sqrt_a, sqrt_oma


_SQRT_A, _SQRT_OMA = _schedule_tables()


_BB = 1


_SPLIT = 2


def _fma_body(ts_ref, ta_ref, tb_ref, x_ref, n_ref, o_ref):
    i = pl.program_id(0)
    t = ts_ref[i // _SPLIT]
    a = ta_ref[t]
    s = tb_ref[t]
    o_ref[...] = x_ref[...] * a + n_ref[...] * s


def kernel(x_0, noise, timesteps):
    B, N, C = x_0.shape
    grid = (B * _SPLIT,)
    spec = pl.BlockSpec((1, N // _SPLIT, C),
                        lambda i, *_: (i // _SPLIT, i % _SPLIT, 0))
    return pl.pallas_call(
        _fma_body,
        grid_spec=pltpu.PrefetchScalarGridSpec(
            num_scalar_prefetch=3,
            grid=grid,
            in_specs=[spec, spec],
            out_specs=spec,
        ),
        out_shape=jax.ShapeDtypeStruct((B, N, C), x_0.dtype),
        compiler_params=pltpu.CompilerParams(
            dimension_semantics=("parallel",),
        ),
    )(timesteps.astype(jnp.int32), jnp.asarray(_SQRT_A), jnp.asarray(_SQRT_OMA),
      x_0, noise)


# speedup vs baseline: 1.0767x; 1.0175x over previous
import numpy as np

import jax
import jax.numpy as jnp
from jax.experimental import pallas as pl
from jax.experimental.pallas import tpu as pltpu

_NUM_TRAIN_TIMESTEPS = 1000
_BETA_START = 0.0001
_BETA_END = 0.02


def _schedule_tables():
    betas = np.linspace(_BETA_START, _BETA_END, _NUM_TRAIN_TIMESTEPS,
                        dtype=np.float32)
    alphas_cumprod = np.cumprod(1.0 - betas, axis=0, dtype=np.float32)
    sqrt_a = np.sqrt(alphas_cumprod).astype(np.float32)
    sqrt_oma = np.sqrt(1.0 - alphas_cumprod).astype(np.float32)
    return sqrt_a, sqrt_oma


_SQRT_A, _SQRT_OMA = _schedule_tables()


_BB = 1


def _fma_body(ts_ref, ta_ref, tb_ref, x_ref, n_ref, o_ref):
    i = pl.program_id(0)
    for k in range(_BB):
        t = ts_ref[_BB * i + k]
        a = ta_ref[t]
        s = tb_ref[t]
        o_ref[k] = x_ref[k] * a + n_ref[k] * s


def kernel(x_0, noise, timesteps):
    B, N, C = x_0.shape
    grid = (B // _BB,)
    spec = pl.BlockSpec((_BB, N, C), lambda i, *_: (i, 0, 0))
    return pl.pallas_call(
        _fma_body,
        grid_spec=pltpu.PrefetchScalarGridSpec(
            num_scalar_prefetch=3,
            grid=grid,
            in_specs=[spec, spec],
            out_specs=spec,
        ),
        out_shape=jax.ShapeDtypeStruct((B, N, C), x_0.dtype),
        compiler_params=pltpu.CompilerParams(
            dimension_semantics=("parallel",),
        ),
    )(timesteps.astype(jnp.int32), jnp.asarray(_SQRT_A), jnp.asarray(_SQRT_OMA),
      x_0, noise)
